# Initial kernel scaffold; baseline (speedup 1.0000x reference)
import jax
import jax.numpy as jnp
from jax.experimental import pallas as pl


def kernel(item_indices, embedding_table, pred_W, pred_b):
    raise NotImplementedError("write your pallas kernel here")


# speedup vs baseline: 1.0608x; 1.0608x over previous
import functools

import jax
import jax.numpy as jnp
from jax import lax
from jax.experimental import pallas as pl
from jax.experimental.pallas import tpu as pltpu
from jax.experimental.pallas import tpu_sc as plsc


def _sc_kernel(B, D, L, NC, BPW, GCHUNK):
    mesh = plsc.VectorSubcoreMesh(core_axis_name="c", subcore_axis_name="s")
    n_chunks = BPW // GCHUNK
    blocks_per_chunk = GCHUNK // L

    @functools.partial(
        pl.kernel,
        mesh=mesh,
        out_type=jax.ShapeDtypeStruct((B,), jnp.float32),
        scratch_types=[
            pltpu.VMEM((BPW,), jnp.int32),
            pltpu.VMEM((BPW, D), jnp.float32),
            pltpu.VMEM((D,), jnp.float32),
            pltpu.VMEM((L,), jnp.float32),
            pltpu.VMEM((BPW,), jnp.float32),
            pltpu.SemaphoreType.DMA,
        ],
        compiler_params=pltpu.CompilerParams(needs_layout_passes=False),
    )
    def k(idx_hbm, table_hbm, w_hbm, b_hbm, out_hbm,
          idx_v, rows_v, w_v, b_v, out_v, sem):
        wid = lax.axis_index("s") * NC + lax.axis_index("c")
        base = wid * BPW

        pltpu.sync_copy(w_hbm, w_v)
        pltpu.sync_copy(b_hbm, b_v)
        pltpu.sync_copy(idx_hbm.at[pl.ds(base, BPW)], idx_v)

        copies = []
        for g in range(n_chunks):
            copies.append(pltpu.async_copy(
                table_hbm.at[idx_v.at[pl.ds(g * GCHUNK, GCHUNK)]],
                rows_v.at[pl.ds(g * GCHUNK, GCHUNK), :],
                sem,
            ))

        bv = b_v[...]
        lane = lax.iota(jnp.int32, L)
        zero = jnp.zeros((L,), jnp.int32)
        zf = jnp.zeros((L,), jnp.float32)

        for g in range(n_chunks):
            copies[g].wait()
            base_row = g * GCHUNK

            def d_body(d, accs, base_row=base_row):
                dd = zero + d
                wd = plsc.load_gather(w_v, [dd])
                new = []
                for blk in range(blocks_per_chunk):
                    ridx = lane + (base_row + blk * L)
                    col = plsc.load_gather(rows_v, [ridx, dd])
                    new.append(accs[blk] + col * wd)
                return tuple(new)

            accs = lax.fori_loop(
                0, D, d_body, tuple(zf for _ in range(blocks_per_chunk)))
            for blk in range(blocks_per_chunk):
                logit = accs[blk] + bv
                rating = 1.0 / (1.0 + jnp.exp(-logit))
                out_v[pl.ds(base_row + blk * L, L)] = rating

        pltpu.sync_copy(out_v, out_hbm.at[pl.ds(base, BPW)])

    return k


def kernel(item_indices, embedding_table, pred_W, pred_b):
    B = item_indices.shape[0]
    V, D = embedding_table.shape
    info = plsc.get_sparse_core_info()
    NC, NS, L = info.num_cores, info.num_subcores, info.num_lanes
    NW = NC * NS
    BPW = B // NW
    GCHUNK = 128

    w_flat = pred_W.reshape(D).astype(jnp.float32)
    b_vec = jnp.broadcast_to(pred_b.astype(jnp.float32), (L,))

    out = _sc_kernel(B, D, L, NC, BPW, GCHUNK)(
        item_indices.astype(jnp.int32), embedding_table, w_flat, b_vec)
    return out.reshape(B, 1)


# speedup vs baseline: 2.2078x; 2.0812x over previous
import functools

import jax
import jax.numpy as jnp
from jax import lax
from jax.experimental import pallas as pl
from jax.experimental.pallas import tpu as pltpu
from jax.experimental.pallas import tpu_sc as plsc


def _sc_kernel(B, D, L, NC, BPW, GCHUNK):
    mesh = plsc.VectorSubcoreMesh(core_axis_name="c", subcore_axis_name="s")
    n_chunks = BPW // GCHUNK

    @functools.partial(
        pl.kernel,
        mesh=mesh,
        out_type=jax.ShapeDtypeStruct((B,), jnp.float32),
        scratch_types=[
            pltpu.VMEM((BPW,), jnp.int32),
            pltpu.VMEM((BPW, D), jnp.float32),
            pltpu.VMEM((D,), jnp.float32),
            pltpu.VMEM((L,), jnp.float32),
            pltpu.VMEM((BPW,), jnp.float32),
            pltpu.SemaphoreType.DMA,
        ],
        compiler_params=pltpu.CompilerParams(needs_layout_passes=False),
    )
    def k(idx_hbm, table_hbm, w_hbm, b_hbm, out_hbm,
          idx_v, rows_v, w_v, b_v, out_v, sem):
        wid = lax.axis_index("s") * NC + lax.axis_index("c")
        base = wid * BPW

        pltpu.sync_copy(w_hbm, w_v)
        pltpu.sync_copy(b_hbm, b_v)
        pltpu.sync_copy(idx_hbm.at[pl.ds(base, BPW)], idx_v)

        copies = []
        for g in range(n_chunks):
            copies.append(pltpu.async_copy(
                table_hbm.at[idx_v.at[pl.ds(g * GCHUNK, GCHUNK)]],
                rows_v.at[pl.ds(g * GCHUNK, GCHUNK), :],
                sem,
            ))

        w_chunks = [w_v[pl.ds(c * L, L)] for c in range(D // L)]
        bscalar = b_v[...]
        lane = lax.iota(jnp.int32, L)
        zero = jnp.zeros((L,), jnp.int32)
        lane0 = lane == 0
        perms = [jnp.arange(L, dtype=jnp.int32) ^ s for s in (8, 4, 2, 1)]

        for g in range(n_chunks):
            copies[g].wait()

            @plsc.parallel_loop(g * GCHUNK, (g + 1) * GCHUNK, unroll=8)
            def row_body(j):
                prods = [rows_v[j, pl.ds(c * L, L)] * w_chunks[c]
                         for c in range(D // L)]
                while len(prods) > 1:
                    prods = [prods[i] + prods[i + 1]
                             for i in range(0, len(prods), 2)]
                s = prods[0]
                for p in perms:
                    s = s + s.at[p].get(mode="promise_in_bounds")
                logit = s + bscalar
                rating = 1.0 / (1.0 + jnp.exp(-logit))
                plsc.store_scatter(out_v, [zero + j], rating, mask=lane0)

        pltpu.sync_copy(out_v, out_hbm.at[pl.ds(base, BPW)])

    return k


def kernel(item_indices, embedding_table, pred_W, pred_b):
    B = item_indices.shape[0]
    V, D = embedding_table.shape
    info = plsc.get_sparse_core_info()
    NC, NS, L = info.num_cores, info.num_subcores, info.num_lanes
    NW = NC * NS
    BPW = B // NW
    GCHUNK = 128

    w_flat = pred_W.reshape(D).astype(jnp.float32)
    b_vec = jnp.broadcast_to(pred_b.astype(jnp.float32), (L,))

    out = _sc_kernel(B, D, L, NC, BPW, GCHUNK)(
        item_indices.astype(jnp.int32), embedding_table, w_flat, b_vec)
    return out.reshape(B, 1)
